# baseline (speedup 1.0000x reference)
import functools

import jax
import jax.numpy as jnp
from jax import lax
from jax.experimental import pallas as pl
from jax.experimental.pallas import tpu as pltpu
from jax.experimental.pallas import tpu_sc as plsc

N_NODES = 10000
FEAT = 128
NUM_HEADS = 2
HF = NUM_HEADS * FEAT
N_RBF = 20
CUTOFF = 5.0
N_EDGES = 160000

_NODE_BLK = 400
_EDGE_BLK = 1600
_NPIECE = 5
_PIECE_E = N_EDGES // _NPIECE
_PIECE_BLKS = _PIECE_E // _EDGE_BLK
_CH = 128
_NW = 32
_PIECE_CHUNKS = _PIECE_E // _CH


def _silu(x):
    return x * jax.nn.sigmoid(x)


def _node_body(s_ref, g_ref, b_ref, wq_ref, wkv_ref, q_ref, kv_ref):
    x = s_ref[...]
    mu = jnp.mean(x, axis=1, keepdims=True)
    xc = x - mu
    var = jnp.mean(xc * xc, axis=1, keepdims=True)
    xn = xc * lax.rsqrt(var + 1e-5) * g_ref[...] + b_ref[...]
    q_ref[...] = jnp.dot(xn, wq_ref[...], preferred_element_type=jnp.float32)
    kv_ref[...] = jnp.dot(xn, wkv_ref[...], preferred_element_type=jnp.float32)


def _edge_body(dist_ref, qi_ref, kvj_ref, wdk_ref, bdk_ref, wdv_ref,
               bdv_ref, wd_ref, bd_ref, prev_ref, out_ref):
    del prev_ref
    d = dist_ref[...]
    lane = lax.broadcasted_iota(jnp.int32, (1, FEAT), 1)
    width = CUTOFF / (N_RBF - 1)
    coeff = -0.5 / (width * width)
    diff = d - lane.astype(jnp.float32) * width
    rbf = jnp.where(lane < N_RBF, jnp.exp(coeff * diff * diff), 0.0)
    dk = _silu(jnp.dot(rbf, wdk_ref[...], preferred_element_type=jnp.float32)
               + bdk_ref[...])
    dv = _silu(jnp.dot(rbf, wdv_ref[...], preferred_element_type=jnp.float32)
               + bdv_ref[...])
    kvj = kvj_ref[...]
    t = qi_ref[...] * kvj[:, :HF] * dk
    a0 = _silu(jnp.sum(t[:, :FEAT], axis=1, keepdims=True))
    a1 = _silu(jnp.sum(t[:, FEAT:], axis=1, keepdims=True))
    w = kvj[:, HF:] * dv
    msg = jnp.concatenate([w[:, :FEAT] * a0, w[:, FEAT:] * a1], axis=1)
    out_ref[...] = (jnp.dot(msg, wd_ref[...], preferred_element_type=jnp.float32)
                    + bd_ref[...])


def _sc_gather_body(piece, q_hbm, kv_hbm, nbrs_hbm, qi_hbm, kvj_hbm,
                    nb_v, ii_v, jj_v, bq, bkv, sem):
    nc = 2
    wid = lax.axis_index("s") * nc + lax.axis_index("c")
    piece_base = piece * _PIECE_E

    def body(t, carry):
        c = wid + t * _NW

        @pl.when(c < _PIECE_CHUNKS)
        def _():
            base = piece_base + c * _CH
            pltpu.sync_copy(nbrs_hbm.at[pl.ds(2 * base, 2 * _CH)], nb_v)
            lanes = lax.iota(jnp.int32, 16)
            for g in range(_CH // 16):
                idx = lanes * 2 + (g * 32)
                ii_v[pl.ds(g * 16, 16)] = plsc.load_gather(nb_v, [idx])
                jj_v[pl.ds(g * 16, 16)] = plsc.load_gather(nb_v, [idx + 1])
            c1 = pltpu.async_copy(q_hbm.at[ii_v], bq, sem)
            c2 = pltpu.async_copy(kv_hbm.at[jj_v], bkv, sem)
            c1.wait()
            c2.wait()
            lbase = c * _CH
            pltpu.sync_copy(bq, qi_hbm.at[pl.ds(lbase, _CH)])
            pltpu.sync_copy(bkv, kvj_hbm.at[pl.ds(lbase, _CH)])

        return carry

    iters = (_PIECE_CHUNKS + _NW - 1) // _NW
    lax.fori_loop(0, iters, body, None)


def _project_nodes(s_j, ln_g, ln_b, Wq, Wkv):
    full = lambda shape: pl.BlockSpec(shape, lambda i: (0, 0))
    return pl.pallas_call(
        _node_body,
        grid=(N_NODES // _NODE_BLK,),
        in_specs=[
            pl.BlockSpec((_NODE_BLK, FEAT), lambda i: (i, 0)),
            full((1, FEAT)),
            full((1, FEAT)),
            full((FEAT, HF)),
            full((FEAT, 2 * HF)),
        ],
        out_specs=[
            pl.BlockSpec((_NODE_BLK, HF), lambda i: (i, 0)),
            pl.BlockSpec((_NODE_BLK, 2 * HF), lambda i: (i, 0)),
        ],
        out_shape=[
            jax.ShapeDtypeStruct((N_NODES, HF), jnp.float32),
            jax.ShapeDtypeStruct((N_NODES, 2 * HF), jnp.float32),
        ],
    )(s_j, ln_g.reshape(1, FEAT), ln_b.reshape(1, FEAT), Wq, Wkv)


def _gather_piece(piece, q, kv, nbrs_flat):
    mesh = plsc.VectorSubcoreMesh(core_axis_name="c", subcore_axis_name="s")
    call = functools.partial(
        pl.kernel,
        mesh=mesh,
        compiler_params=pltpu.CompilerParams(needs_layout_passes=False),
        out_type=[
            jax.ShapeDtypeStruct((_PIECE_E, HF), jnp.float32),
            jax.ShapeDtypeStruct((_PIECE_E, 2 * HF), jnp.float32),
        ],
        scratch_types=[
            pltpu.VMEM((2 * _CH,), jnp.int32),
            pltpu.VMEM((_CH,), jnp.int32),
            pltpu.VMEM((_CH,), jnp.int32),
            pltpu.VMEM((_CH, HF), jnp.float32),
            pltpu.VMEM((_CH, 2 * HF), jnp.float32),
            pltpu.SemaphoreType.DMA,
        ],
    )(functools.partial(_sc_gather_body, piece))
    return call(q, kv, nbrs_flat)


def _edge_compute_piece(piece, dist2, qi_p, kvj_p, Wdk_p, bdk, Wdv_p, bdv,
                        Wd, bd, prev_out):
    full = lambda shape: pl.BlockSpec(shape, lambda i: (0, 0))
    off = piece * _PIECE_BLKS
    return pl.pallas_call(
        _edge_body,
        grid=(_PIECE_BLKS,),
        in_specs=[
            pl.BlockSpec((_EDGE_BLK, 1), lambda i: (off + i, 0)),
            pl.BlockSpec((_EDGE_BLK, HF), lambda i: (i, 0)),
            pl.BlockSpec((_EDGE_BLK, 2 * HF), lambda i: (i, 0)),
            full((FEAT, HF)),
            full((1, HF)),
            full((FEAT, HF)),
            full((1, HF)),
            full((HF, 3 * FEAT)),
            full((1, 3 * FEAT)),
            pl.BlockSpec(memory_space=pl.ANY),
        ],
        out_specs=pl.BlockSpec((_EDGE_BLK, 3 * FEAT), lambda i: (off + i, 0)),
        out_shape=jax.ShapeDtypeStruct((N_EDGES, 3 * FEAT), jnp.float32),
        input_output_aliases={} if prev_out is None else {9: 0},
    )(dist2, qi_p, kvj_p, Wdk_p, bdk.reshape(1, HF), Wdv_p,
      bdv.reshape(1, HF), Wd, bd.reshape(1, 3 * FEAT),
      jnp.zeros((8, 128), jnp.float32) if prev_out is None else prev_out)


def kernel(s_j, dist, nbrs, ln_g, ln_b, Wq, Wk, Wv, Wdk, bdk, Wdv, bdv, Wd, bd):
    Wkv = jnp.concatenate([Wk, Wv], axis=1)
    q, kv = _project_nodes(s_j, ln_g, ln_b, Wq, Wkv)
    nbrs_flat = nbrs.astype(jnp.int32).reshape(2 * N_EDGES)
    Wdk_p = jnp.zeros((FEAT, HF), jnp.float32).at[:N_RBF].set(Wdk)
    Wdv_p = jnp.zeros((FEAT, HF), jnp.float32).at[:N_RBF].set(Wdv)
    dist2 = dist.reshape(N_EDGES, 1)

    gathered = [_gather_piece(p, q, kv, nbrs_flat) for p in range(_NPIECE)]
    out = None
    for p in range(_NPIECE):
        qi_p, kvj_p = gathered[p]
        out = _edge_compute_piece(p, dist2, qi_p, kvj_p, Wdk_p, bdk,
                                  Wdv_p, bdv, Wd, bd, out)
    return out.reshape(N_EDGES, 3, FEAT)


# speedup vs baseline: 1.1662x; 1.1662x over previous
import functools

import jax
import jax.numpy as jnp
from jax import lax
from jax.experimental import pallas as pl
from jax.experimental.pallas import tpu as pltpu
from jax.experimental.pallas import tpu_sc as plsc

N_NODES = 10000
FEAT = 128
NUM_HEADS = 2
HF = NUM_HEADS * FEAT
N_RBF = 20
CUTOFF = 5.0
N_EDGES = 160000

_NODE_BLK = 400
_EDGE_BLK = 1600
_NPIECE = 5
_PIECE_E = N_EDGES // _NPIECE
_PIECE_BLKS = _PIECE_E // _EDGE_BLK
_CH = 128
_NW = 32
_PIECE_CHUNKS = _PIECE_E // _CH


def _silu(x):
    return x * jax.nn.sigmoid(x)


def _node_body(s_ref, g_ref, b_ref, wq_ref, wkv_ref, q_ref, kv_ref):
    x = s_ref[...]
    mu = jnp.mean(x, axis=1, keepdims=True)
    xc = x - mu
    var = jnp.mean(xc * xc, axis=1, keepdims=True)
    xn = xc * lax.rsqrt(var + 1e-5) * g_ref[...] + b_ref[...]
    q_ref[...] = jnp.dot(xn, wq_ref[...], preferred_element_type=jnp.float32)
    kv_ref[...] = jnp.dot(xn, wkv_ref[...], preferred_element_type=jnp.float32)


def _edge_body(dist_ref, qi_ref, kvj_ref, wdk_ref, bdk_ref, wdv_ref,
               bdv_ref, wd_ref, bd_ref, prev_ref, out_ref):
    del prev_ref
    d = dist_ref[...]
    lane = lax.broadcasted_iota(jnp.int32, (1, FEAT), 1)
    width = CUTOFF / (N_RBF - 1)
    coeff = -0.5 / (width * width)
    diff = d - lane.astype(jnp.float32) * width
    rbf = jnp.where(lane < N_RBF, jnp.exp(coeff * diff * diff), 0.0)
    dk = _silu(jnp.dot(rbf, wdk_ref[...], preferred_element_type=jnp.float32)
               + bdk_ref[...])
    dv = _silu(jnp.dot(rbf, wdv_ref[...], preferred_element_type=jnp.float32)
               + bdv_ref[...])
    kvj = kvj_ref[...]
    t = qi_ref[...] * kvj[:, :HF] * dk
    a0 = _silu(jnp.sum(t[:, :FEAT], axis=1, keepdims=True))
    a1 = _silu(jnp.sum(t[:, FEAT:], axis=1, keepdims=True))
    w = kvj[:, HF:] * dv
    msg = jnp.concatenate([w[:, :FEAT] * a0, w[:, FEAT:] * a1], axis=1)
    res = (jnp.dot(msg, wd_ref[...], preferred_element_type=jnp.float32)
           + bd_ref[...])
    out_ref[:, 0, :] = res[:, :FEAT]
    out_ref[:, 1, :] = res[:, FEAT:2 * FEAT]
    out_ref[:, 2, :] = res[:, 2 * FEAT:]


def _sc_gather_body(piece, q_hbm, kv_hbm, nbrs_hbm, qi_hbm, kvj_hbm,
                    nb_v, ii_v, jj_v, bq, bkv, sem):
    nc = 2
    wid = lax.axis_index("s") * nc + lax.axis_index("c")
    piece_base = piece * _PIECE_E

    def body(t, carry):
        c = wid + t * _NW

        @pl.when(c < _PIECE_CHUNKS)
        def _():
            base = piece_base + c * _CH
            pltpu.sync_copy(nbrs_hbm.at[pl.ds(2 * base, 2 * _CH)], nb_v)
            lanes = lax.iota(jnp.int32, 16)
            for g in range(_CH // 16):
                idx = lanes * 2 + (g * 32)
                ii_v[pl.ds(g * 16, 16)] = plsc.load_gather(nb_v, [idx])
                jj_v[pl.ds(g * 16, 16)] = plsc.load_gather(nb_v, [idx + 1])
            c1 = pltpu.async_copy(q_hbm.at[ii_v], bq, sem)
            c2 = pltpu.async_copy(kv_hbm.at[jj_v], bkv, sem)
            c1.wait()
            c2.wait()
            lbase = c * _CH
            pltpu.sync_copy(bq, qi_hbm.at[pl.ds(lbase, _CH)])
            pltpu.sync_copy(bkv, kvj_hbm.at[pl.ds(lbase, _CH)])

        return carry

    iters = (_PIECE_CHUNKS + _NW - 1) // _NW
    lax.fori_loop(0, iters, body, None)


def _project_nodes(s_j, ln_g, ln_b, Wq, Wkv):
    full = lambda shape: pl.BlockSpec(shape, lambda i: (0, 0))
    return pl.pallas_call(
        _node_body,
        grid=(N_NODES // _NODE_BLK,),
        in_specs=[
            pl.BlockSpec((_NODE_BLK, FEAT), lambda i: (i, 0)),
            full((1, FEAT)),
            full((1, FEAT)),
            full((FEAT, HF)),
            full((FEAT, 2 * HF)),
        ],
        out_specs=[
            pl.BlockSpec((_NODE_BLK, HF), lambda i: (i, 0)),
            pl.BlockSpec((_NODE_BLK, 2 * HF), lambda i: (i, 0)),
        ],
        out_shape=[
            jax.ShapeDtypeStruct((N_NODES, HF), jnp.float32),
            jax.ShapeDtypeStruct((N_NODES, 2 * HF), jnp.float32),
        ],
    )(s_j, ln_g.reshape(1, FEAT), ln_b.reshape(1, FEAT), Wq, Wkv)


def _gather_piece(piece, q, kv, nbrs_flat):
    mesh = plsc.VectorSubcoreMesh(core_axis_name="c", subcore_axis_name="s")
    call = functools.partial(
        pl.kernel,
        mesh=mesh,
        compiler_params=pltpu.CompilerParams(needs_layout_passes=False),
        out_type=[
            jax.ShapeDtypeStruct((_PIECE_E, HF), jnp.float32),
            jax.ShapeDtypeStruct((_PIECE_E, 2 * HF), jnp.float32),
        ],
        scratch_types=[
            pltpu.VMEM((2 * _CH,), jnp.int32),
            pltpu.VMEM((_CH,), jnp.int32),
            pltpu.VMEM((_CH,), jnp.int32),
            pltpu.VMEM((_CH, HF), jnp.float32),
            pltpu.VMEM((_CH, 2 * HF), jnp.float32),
            pltpu.SemaphoreType.DMA,
        ],
    )(functools.partial(_sc_gather_body, piece))
    return call(q, kv, nbrs_flat)


def _edge_compute_piece(piece, dist2, qi_p, kvj_p, Wdk_p, bdk, Wdv_p, bdv,
                        Wd, bd, prev_out):
    full = lambda shape: pl.BlockSpec(shape, lambda i: (0, 0))
    off = piece * _PIECE_BLKS
    return pl.pallas_call(
        _edge_body,
        grid=(_PIECE_BLKS,),
        in_specs=[
            pl.BlockSpec((_EDGE_BLK, 1), lambda i: (off + i, 0)),
            pl.BlockSpec((_EDGE_BLK, HF), lambda i: (i, 0)),
            pl.BlockSpec((_EDGE_BLK, 2 * HF), lambda i: (i, 0)),
            full((FEAT, HF)),
            full((1, HF)),
            full((FEAT, HF)),
            full((1, HF)),
            full((HF, 3 * FEAT)),
            full((1, 3 * FEAT)),
            pl.BlockSpec(memory_space=pl.ANY),
        ],
        out_specs=pl.BlockSpec((_EDGE_BLK, 3, FEAT), lambda i: (off + i, 0, 0)),
        out_shape=jax.ShapeDtypeStruct((N_EDGES, 3, FEAT), jnp.float32),
        input_output_aliases={} if prev_out is None else {9: 0},
    )(dist2, qi_p, kvj_p, Wdk_p, bdk.reshape(1, HF), Wdv_p,
      bdv.reshape(1, HF), Wd, bd.reshape(1, 3 * FEAT),
      jnp.zeros((8, 128), jnp.float32) if prev_out is None else prev_out)


def kernel(s_j, dist, nbrs, ln_g, ln_b, Wq, Wk, Wv, Wdk, bdk, Wdv, bdv, Wd, bd):
    Wkv = jnp.concatenate([Wk, Wv], axis=1)
    q, kv = _project_nodes(s_j, ln_g, ln_b, Wq, Wkv)
    nbrs_flat = nbrs.astype(jnp.int32).reshape(2 * N_EDGES)
    Wdk_p = jnp.zeros((FEAT, HF), jnp.float32).at[:N_RBF].set(Wdk)
    Wdv_p = jnp.zeros((FEAT, HF), jnp.float32).at[:N_RBF].set(Wdv)
    dist2 = dist.reshape(N_EDGES, 1)

    gathered = [_gather_piece(p, q, kv, nbrs_flat) for p in range(_NPIECE)]
    out = None
    for p in range(_NPIECE):
        qi_p, kvj_p = gathered[p]
        out = _edge_compute_piece(p, dist2, qi_p, kvj_p, Wdk_p, bdk,
                                  Wdv_p, bdv, Wd, bd, out)
    return out


# speedup vs baseline: 1.4703x; 1.2608x over previous
import functools

import jax
import jax.numpy as jnp
from jax import lax
from jax.experimental import pallas as pl
from jax.experimental.pallas import tpu as pltpu
from jax.experimental.pallas import tpu_sc as plsc

N_NODES = 10000
FEAT = 128
NUM_HEADS = 2
HF = NUM_HEADS * FEAT
N_RBF = 20
CUTOFF = 5.0
N_EDGES = 160000

_NODE_BLK = 400
_EDGE_BLK = 1600
_NPIECE = 5
_PIECE_E = N_EDGES // _NPIECE
_PIECE_BLKS = _PIECE_E // _EDGE_BLK
_CH = 128
_NW = 32
_PIECE_CHUNKS = _PIECE_E // _CH


def _silu(x):
    return x * jax.nn.sigmoid(x)


def _node_body(s_ref, g_ref, b_ref, wq_ref, wkv_ref, q_ref, kv_ref):
    x = s_ref[...]
    mu = jnp.mean(x, axis=1, keepdims=True)
    xc = x - mu
    var = jnp.mean(xc * xc, axis=1, keepdims=True)
    xn = xc * lax.rsqrt(var + 1e-5) * g_ref[...] + b_ref[...]
    q_ref[...] = jnp.dot(xn, wq_ref[...], preferred_element_type=jnp.float32)
    kv_ref[...] = jnp.dot(xn, wkv_ref[...], preferred_element_type=jnp.float32)


def _edge_body(dist_ref, qi_ref, kvj_ref, wdk_ref, bdk_ref, wdv_ref,
               bdv_ref, wd_ref, bd_ref, prev_ref, out_ref):
    del prev_ref
    d = dist_ref[...]
    lane = lax.broadcasted_iota(jnp.int32, (1, FEAT), 1)
    width = CUTOFF / (N_RBF - 1)
    coeff = -0.5 / (width * width)
    diff = d - lane.astype(jnp.float32) * width
    rbf = jnp.where(lane < N_RBF, jnp.exp(coeff * diff * diff), 0.0)
    dk = _silu(jnp.dot(rbf, wdk_ref[...], preferred_element_type=jnp.float32)
               + bdk_ref[...])
    dv = _silu(jnp.dot(rbf, wdv_ref[...], preferred_element_type=jnp.float32)
               + bdv_ref[...])
    kvj = kvj_ref[...]
    t = qi_ref[...] * kvj[:, :HF] * dk
    a0 = _silu(jnp.sum(t[:, :FEAT], axis=1, keepdims=True))
    a1 = _silu(jnp.sum(t[:, FEAT:], axis=1, keepdims=True))
    w = kvj[:, HF:] * dv
    msg = jnp.concatenate([w[:, :FEAT] * a0, w[:, FEAT:] * a1], axis=1)
    res = (jnp.dot(msg, wd_ref[...], preferred_element_type=jnp.float32)
           + bd_ref[...])
    out_ref[0, :, :] = res[:, :FEAT]
    out_ref[1, :, :] = res[:, FEAT:2 * FEAT]
    out_ref[2, :, :] = res[:, 2 * FEAT:]


def _sc_gather_body(piece, q_hbm, kv_hbm, nbrs_hbm, qi_hbm, kvj_hbm,
                    nb_v, ii_v, jj_v, bq, bkv, sem):
    nc = 2
    wid = lax.axis_index("s") * nc + lax.axis_index("c")
    piece_base = piece * _PIECE_E

    def body(t, carry):
        c = wid + t * _NW

        @pl.when(c < _PIECE_CHUNKS)
        def _():
            base = piece_base + c * _CH
            pltpu.sync_copy(nbrs_hbm.at[pl.ds(2 * base, 2 * _CH)], nb_v)
            lanes = lax.iota(jnp.int32, 16)
            for g in range(_CH // 16):
                idx = lanes * 2 + (g * 32)
                ii_v[pl.ds(g * 16, 16)] = plsc.load_gather(nb_v, [idx])
                jj_v[pl.ds(g * 16, 16)] = plsc.load_gather(nb_v, [idx + 1])
            c1 = pltpu.async_copy(q_hbm.at[ii_v], bq, sem)
            c2 = pltpu.async_copy(kv_hbm.at[jj_v], bkv, sem)
            c1.wait()
            c2.wait()
            lbase = c * _CH
            pltpu.sync_copy(bq, qi_hbm.at[pl.ds(lbase, _CH)])
            pltpu.sync_copy(bkv, kvj_hbm.at[pl.ds(lbase, _CH)])

        return carry

    iters = (_PIECE_CHUNKS + _NW - 1) // _NW
    lax.fori_loop(0, iters, body, None)


def _project_nodes(s_j, ln_g, ln_b, Wq, Wkv):
    full = lambda shape: pl.BlockSpec(shape, lambda i: (0, 0))
    return pl.pallas_call(
        _node_body,
        grid=(N_NODES // _NODE_BLK,),
        in_specs=[
            pl.BlockSpec((_NODE_BLK, FEAT), lambda i: (i, 0)),
            full((1, FEAT)),
            full((1, FEAT)),
            full((FEAT, HF)),
            full((FEAT, 2 * HF)),
        ],
        out_specs=[
            pl.BlockSpec((_NODE_BLK, HF), lambda i: (i, 0)),
            pl.BlockSpec((_NODE_BLK, 2 * HF), lambda i: (i, 0)),
        ],
        out_shape=[
            jax.ShapeDtypeStruct((N_NODES, HF), jnp.float32),
            jax.ShapeDtypeStruct((N_NODES, 2 * HF), jnp.float32),
        ],
    )(s_j, ln_g.reshape(1, FEAT), ln_b.reshape(1, FEAT), Wq, Wkv)


def _gather_piece(piece, q, kv, nbrs_flat):
    mesh = plsc.VectorSubcoreMesh(core_axis_name="c", subcore_axis_name="s")
    call = functools.partial(
        pl.kernel,
        mesh=mesh,
        compiler_params=pltpu.CompilerParams(needs_layout_passes=False),
        out_type=[
            jax.ShapeDtypeStruct((_PIECE_E, HF), jnp.float32),
            jax.ShapeDtypeStruct((_PIECE_E, 2 * HF), jnp.float32),
        ],
        scratch_types=[
            pltpu.VMEM((2 * _CH,), jnp.int32),
            pltpu.VMEM((_CH,), jnp.int32),
            pltpu.VMEM((_CH,), jnp.int32),
            pltpu.VMEM((_CH, HF), jnp.float32),
            pltpu.VMEM((_CH, 2 * HF), jnp.float32),
            pltpu.SemaphoreType.DMA,
        ],
    )(functools.partial(_sc_gather_body, piece))
    return call(q, kv, nbrs_flat)


def _edge_compute_piece(piece, dist2, qi_p, kvj_p, Wdk_p, bdk, Wdv_p, bdv,
                        Wd, bd, prev_out):
    full = lambda shape: pl.BlockSpec(shape, lambda i: (0, 0))
    off = piece * _PIECE_BLKS
    return pl.pallas_call(
        _edge_body,
        grid=(_PIECE_BLKS,),
        in_specs=[
            pl.BlockSpec((_EDGE_BLK, 1), lambda i: (off + i, 0)),
            pl.BlockSpec((_EDGE_BLK, HF), lambda i: (i, 0)),
            pl.BlockSpec((_EDGE_BLK, 2 * HF), lambda i: (i, 0)),
            full((FEAT, HF)),
            full((1, HF)),
            full((FEAT, HF)),
            full((1, HF)),
            full((HF, 3 * FEAT)),
            full((1, 3 * FEAT)),
            pl.BlockSpec(memory_space=pl.ANY),
        ],
        out_specs=pl.BlockSpec((3, _EDGE_BLK, FEAT), lambda i: (0, off + i, 0)),
        out_shape=jax.ShapeDtypeStruct((3, N_EDGES, FEAT), jnp.float32),
        input_output_aliases={} if prev_out is None else {9: 0},
    )(dist2, qi_p, kvj_p, Wdk_p, bdk.reshape(1, HF), Wdv_p,
      bdv.reshape(1, HF), Wd, bd.reshape(1, 3 * FEAT),
      jnp.zeros((8, 128), jnp.float32) if prev_out is None else prev_out)


def kernel(s_j, dist, nbrs, ln_g, ln_b, Wq, Wk, Wv, Wdk, bdk, Wdv, bdv, Wd, bd):
    Wkv = jnp.concatenate([Wk, Wv], axis=1)
    q, kv = _project_nodes(s_j, ln_g, ln_b, Wq, Wkv)
    nbrs_flat = nbrs.astype(jnp.int32).reshape(2 * N_EDGES)
    Wdk_p = jnp.zeros((FEAT, HF), jnp.float32).at[:N_RBF].set(Wdk)
    Wdv_p = jnp.zeros((FEAT, HF), jnp.float32).at[:N_RBF].set(Wdv)
    dist2 = dist.reshape(N_EDGES, 1)

    gathered = [_gather_piece(p, q, kv, nbrs_flat) for p in range(_NPIECE)]
    out = None
    for p in range(_NPIECE):
        qi_p, kvj_p = gathered[p]
        out = _edge_compute_piece(p, dist2, qi_p, kvj_p, Wdk_p, bdk,
                                  Wdv_p, bdv, Wd, bd, out)
    return jnp.transpose(out, (1, 0, 2))


# speedup vs baseline: 1.6522x; 1.1237x over previous
import functools

import jax
import jax.numpy as jnp
from jax import lax
from jax.experimental import pallas as pl
from jax.experimental.pallas import tpu as pltpu
from jax.experimental.pallas import tpu_sc as plsc

N_NODES = 10000
FEAT = 128
NUM_HEADS = 2
HF = NUM_HEADS * FEAT
N_RBF = 20
CUTOFF = 5.0
N_EDGES = 160000

_NODE_BLK = 400
_EDGE_BLK = 1600
_CH = 128
_NW = 32
_NCHUNK = N_EDGES // _CH
_PIECES = (38400, 38400, 38400, 32000, 12800)
assert sum(_PIECES) == N_EDGES


def _silu(x):
    return x * jax.nn.sigmoid(x)


def _node_body(s_ref, g_ref, b_ref, wq_ref, wkv_ref, q_ref, kv_ref):
    x = s_ref[...]
    mu = jnp.mean(x, axis=1, keepdims=True)
    xc = x - mu
    var = jnp.mean(xc * xc, axis=1, keepdims=True)
    xn = xc * lax.rsqrt(var + 1e-5) * g_ref[...] + b_ref[...]
    q_ref[...] = jnp.dot(xn, wq_ref[...], preferred_element_type=jnp.float32)
    kv_ref[...] = jnp.dot(xn, wkv_ref[...], preferred_element_type=jnp.float32)


def _edge_body(dist_ref, qi_ref, kvj_ref, wdk_ref, bdk_ref, wdv_ref,
               bdv_ref, wd_ref, bd_ref, prev_ref, out_ref):
    del prev_ref
    d = dist_ref[...]
    lane = lax.broadcasted_iota(jnp.int32, (1, FEAT), 1)
    width = CUTOFF / (N_RBF - 1)
    coeff = -0.5 / (width * width)
    diff = d - lane.astype(jnp.float32) * width
    rbf = jnp.where(lane < N_RBF, jnp.exp(coeff * diff * diff), 0.0)
    dk = _silu(jnp.dot(rbf, wdk_ref[...], preferred_element_type=jnp.float32)
               + bdk_ref[...])
    dv = _silu(jnp.dot(rbf, wdv_ref[...], preferred_element_type=jnp.float32)
               + bdv_ref[...])
    kvj = kvj_ref[...]
    t = qi_ref[...] * kvj[:, :HF] * dk
    a0 = _silu(jnp.sum(t[:, :FEAT], axis=1, keepdims=True))
    a1 = _silu(jnp.sum(t[:, FEAT:], axis=1, keepdims=True))
    w = kvj[:, HF:] * dv
    msg = jnp.concatenate([w[:, :FEAT] * a0, w[:, FEAT:] * a1], axis=1)
    res = (jnp.dot(msg, wd_ref[...], preferred_element_type=jnp.float32)
           + bd_ref[...])
    out_ref[0, :, :] = res[:, :FEAT]
    out_ref[1, :, :] = res[:, FEAT:2 * FEAT]
    out_ref[2, :, :] = res[:, 2 * FEAT:]


def _sc_gather_body(chunk0, nchunks, q_hbm, kv_hbm, nbrs_hbm, qi_hbm, kvj_hbm,
                    nb_v, bq, bkv, sem):
    nc = 2
    wid = lax.axis_index("s") * nc + lax.axis_index("c")

    def body(t, carry):
        c = wid + t * _NW

        @pl.when(c < nchunks)
        def _():
            pltpu.sync_copy(nbrs_hbm.at[chunk0 + c], nb_v)
            c1 = pltpu.async_copy(q_hbm.at[nb_v.at[0]], bq, sem)
            c2 = pltpu.async_copy(kv_hbm.at[nb_v.at[1]], bkv, sem)
            c1.wait()
            c2.wait()
            lbase = c * _CH
            pltpu.sync_copy(bq, qi_hbm.at[pl.ds(lbase, _CH)])
            pltpu.sync_copy(bkv, kvj_hbm.at[pl.ds(lbase, _CH)])

        return carry

    iters = (nchunks + _NW - 1) // _NW
    lax.fori_loop(0, iters, body, None)


def _project_nodes(s_j, ln_g, ln_b, Wq, Wkv):
    full = lambda shape: pl.BlockSpec(shape, lambda i: (0, 0))
    return pl.pallas_call(
        _node_body,
        grid=(N_NODES // _NODE_BLK,),
        in_specs=[
            pl.BlockSpec((_NODE_BLK, FEAT), lambda i: (i, 0)),
            full((1, FEAT)),
            full((1, FEAT)),
            full((FEAT, HF)),
            full((FEAT, 2 * HF)),
        ],
        out_specs=[
            pl.BlockSpec((_NODE_BLK, HF), lambda i: (i, 0)),
            pl.BlockSpec((_NODE_BLK, 2 * HF), lambda i: (i, 0)),
        ],
        out_shape=[
            jax.ShapeDtypeStruct((N_NODES, HF), jnp.float32),
            jax.ShapeDtypeStruct((N_NODES, 2 * HF), jnp.float32),
        ],
    )(s_j, ln_g.reshape(1, FEAT), ln_b.reshape(1, FEAT), Wq, Wkv)


def _gather_piece(off, size, q, kv, nbrs_chunks):
    mesh = plsc.VectorSubcoreMesh(core_axis_name="c", subcore_axis_name="s")
    call = functools.partial(
        pl.kernel,
        mesh=mesh,
        compiler_params=pltpu.CompilerParams(needs_layout_passes=False),
        out_type=[
            jax.ShapeDtypeStruct((size, HF), jnp.float32),
            jax.ShapeDtypeStruct((size, 2 * HF), jnp.float32),
        ],
        scratch_types=[
            pltpu.VMEM((2, _CH), jnp.int32),
            pltpu.VMEM((_CH, HF), jnp.float32),
            pltpu.VMEM((_CH, 2 * HF), jnp.float32),
            pltpu.SemaphoreType.DMA,
        ],
    )(functools.partial(_sc_gather_body, off // _CH, size // _CH))
    return call(q, kv, nbrs_chunks)


def _edge_compute_piece(off, size, dist2, qi_p, kvj_p, Wdk_p, bdk, Wdv_p, bdv,
                        Wd, bd, prev_out):
    full = lambda shape: pl.BlockSpec(shape, lambda i: (0, 0))
    boff = off // _EDGE_BLK
    return pl.pallas_call(
        _edge_body,
        grid=(size // _EDGE_BLK,),
        in_specs=[
            pl.BlockSpec((_EDGE_BLK, 1), lambda i: (boff + i, 0)),
            pl.BlockSpec((_EDGE_BLK, HF), lambda i: (i, 0)),
            pl.BlockSpec((_EDGE_BLK, 2 * HF), lambda i: (i, 0)),
            full((FEAT, HF)),
            full((1, HF)),
            full((FEAT, HF)),
            full((1, HF)),
            full((HF, 3 * FEAT)),
            full((1, 3 * FEAT)),
            pl.BlockSpec(memory_space=pl.ANY),
        ],
        out_specs=pl.BlockSpec((3, _EDGE_BLK, FEAT), lambda i: (0, boff + i, 0)),
        out_shape=jax.ShapeDtypeStruct((3, N_EDGES, FEAT), jnp.float32),
        input_output_aliases={} if prev_out is None else {9: 0},
    )(dist2, qi_p, kvj_p, Wdk_p, bdk.reshape(1, HF), Wdv_p,
      bdv.reshape(1, HF), Wd, bd.reshape(1, 3 * FEAT),
      jnp.zeros((8, 128), jnp.float32) if prev_out is None else prev_out)


def kernel(s_j, dist, nbrs, ln_g, ln_b, Wq, Wk, Wv, Wdk, bdk, Wdv, bdv, Wd, bd):
    Wkv = jnp.concatenate([Wk, Wv], axis=1)
    q, kv = _project_nodes(s_j, ln_g, ln_b, Wq, Wkv)
    nbrs_chunks = jnp.transpose(
        nbrs.astype(jnp.int32).reshape(_NCHUNK, _CH, 2), (0, 2, 1))
    Wdk_p = jnp.zeros((FEAT, HF), jnp.float32).at[:N_RBF].set(Wdk)
    Wdv_p = jnp.zeros((FEAT, HF), jnp.float32).at[:N_RBF].set(Wdv)
    dist2 = dist.reshape(N_EDGES, 1)

    offs = [sum(_PIECES[:i]) for i in range(len(_PIECES))]
    gathered = [_gather_piece(o, s, q, kv, nbrs_chunks)
                for o, s in zip(offs, _PIECES)]
    out = None
    for (o, s), (qi_p, kvj_p) in zip(zip(offs, _PIECES), gathered):
        out = _edge_compute_piece(o, s, dist2, qi_p, kvj_p, Wdk_p, bdk,
                                  Wdv_p, bdv, Wd, bd, out)
    return jnp.transpose(out, (1, 0, 2))


# speedup vs baseline: 2.4549x; 1.4858x over previous
import functools

import jax
import jax.numpy as jnp
from jax import lax
from jax.experimental import pallas as pl
from jax.experimental.pallas import tpu as pltpu
from jax.experimental.pallas import tpu_sc as plsc

N_NODES = 10000
FEAT = 128
NUM_HEADS = 2
HF = NUM_HEADS * FEAT
N_RBF = 20
CUTOFF = 5.0
N_EDGES = 160000

_NODE_BLK = 400
_EDGE_BLK = 1600
_CH = 128
_NW = 32
_NCHUNK = N_EDGES // _CH
_PIECES = (38400, 38400, 38400, 32000, 12800)
assert sum(_PIECES) == N_EDGES


def _silu(x):
    return x * jax.nn.sigmoid(x)


def _pack_rne(ev, od):
    be = lax.bitcast_convert_type(ev, jnp.uint32)
    bo = lax.bitcast_convert_type(od, jnp.uint32)
    re = (be + jnp.uint32(0x7FFF) + ((be >> 16) & jnp.uint32(1))) >> 16
    ro = (bo + jnp.uint32(0x7FFF) + ((bo >> 16) & jnp.uint32(1))) \
        & jnp.uint32(0xFFFF0000)
    return lax.bitcast_convert_type(re | ro, jnp.int32)


def _unpack(x):
    xb = lax.bitcast_convert_type(x, jnp.uint32)
    ev = lax.bitcast_convert_type(xb << 16, jnp.float32)
    od = lax.bitcast_convert_type(xb & jnp.uint32(0xFFFF0000), jnp.float32)
    return ev, od


def _node_body(s_ref, g_ref, b_ref, wqe_ref, wqo_ref, wkve_ref, wkvo_ref,
               q_ref, kv_ref):
    x = s_ref[...]
    mu = jnp.mean(x, axis=1, keepdims=True)
    xc = x - mu
    var = jnp.mean(xc * xc, axis=1, keepdims=True)
    xn = xc * lax.rsqrt(var + 1e-5) * g_ref[...] + b_ref[...]
    dot = lambda w: jnp.dot(xn, w, preferred_element_type=jnp.float32)
    q_ref[...] = _pack_rne(dot(wqe_ref[...]), dot(wqo_ref[...]))
    kv_ref[...] = _pack_rne(dot(wkve_ref[...]), dot(wkvo_ref[...]))


def _edge_body(dist_ref, qi_ref, kvj_ref, wdk_ref, bdk_ref, wdv_ref,
               bdv_ref, wd_ref, bd_ref, prev_ref, out_ref):
    del prev_ref
    d = dist_ref[...]
    lane = lax.broadcasted_iota(jnp.int32, (1, FEAT), 1)
    width = CUTOFF / (N_RBF - 1)
    coeff = -0.5 / (width * width)
    diff = d - lane.astype(jnp.float32) * width
    rbf = jnp.where(lane < N_RBF, jnp.exp(coeff * diff * diff), 0.0)
    dk = _silu(jnp.dot(rbf, wdk_ref[...], preferred_element_type=jnp.float32)
               + bdk_ref[...])
    dv = _silu(jnp.dot(rbf, wdv_ref[...], preferred_element_type=jnp.float32)
               + bdv_ref[...])
    qe, qo = _unpack(qi_ref[...])
    qi = jnp.concatenate([qe, qo], axis=1)
    kve, kvo = _unpack(kvj_ref[...])
    kj = jnp.concatenate([kve[:, :FEAT], kvo[:, :FEAT]], axis=1)
    vj = jnp.concatenate([kve[:, FEAT:], kvo[:, FEAT:]], axis=1)
    li = lax.broadcasted_iota(jnp.int32, (1, HF), 1)
    h0 = ((li // 64) % 2) == 0
    t = qi * kj * dk
    a0 = _silu(jnp.sum(jnp.where(h0, t, 0.0), axis=1, keepdims=True))
    a1 = _silu(jnp.sum(jnp.where(h0, 0.0, t), axis=1, keepdims=True))
    msg = vj * dv * jnp.where(h0, a0, a1)
    res = (jnp.dot(msg, wd_ref[...], preferred_element_type=jnp.float32)
           + bd_ref[...])
    out_ref[0, :, :] = res[:, :FEAT]
    out_ref[1, :, :] = res[:, FEAT:2 * FEAT]
    out_ref[2, :, :] = res[:, 2 * FEAT:]


def _sc_gather_body(chunk0, nchunks, q_hbm, kv_hbm, nbrs_hbm, qi_hbm, kvj_hbm,
                    nb_v, bq, bkv, sem):
    nc = 2
    wid = lax.axis_index("s") * nc + lax.axis_index("c")

    def body(t, carry):
        c = wid + t * _NW

        @pl.when(c < nchunks)
        def _():
            pltpu.sync_copy(nbrs_hbm.at[chunk0 + c], nb_v)
            c1 = pltpu.async_copy(q_hbm.at[nb_v.at[0]], bq, sem)
            c2 = pltpu.async_copy(kv_hbm.at[nb_v.at[1]], bkv, sem)
            c1.wait()
            c2.wait()
            lbase = c * _CH
            pltpu.sync_copy(bq, qi_hbm.at[pl.ds(lbase, _CH)])
            pltpu.sync_copy(bkv, kvj_hbm.at[pl.ds(lbase, _CH)])

        return carry

    iters = (nchunks + _NW - 1) // _NW
    lax.fori_loop(0, iters, body, None)


def _project_nodes(s_j, ln_g, ln_b, Wq_e, Wq_o, Wkv_e, Wkv_o):
    full = lambda shape: pl.BlockSpec(shape, lambda i: (0, 0))
    return pl.pallas_call(
        _node_body,
        grid=(N_NODES // _NODE_BLK,),
        in_specs=[
            pl.BlockSpec((_NODE_BLK, FEAT), lambda i: (i, 0)),
            full((1, FEAT)),
            full((1, FEAT)),
            full((FEAT, HF // 2)),
            full((FEAT, HF // 2)),
            full((FEAT, HF)),
            full((FEAT, HF)),
        ],
        out_specs=[
            pl.BlockSpec((_NODE_BLK, HF // 2), lambda i: (i, 0)),
            pl.BlockSpec((_NODE_BLK, HF), lambda i: (i, 0)),
        ],
        out_shape=[
            jax.ShapeDtypeStruct((N_NODES, HF // 2), jnp.int32),
            jax.ShapeDtypeStruct((N_NODES, HF), jnp.int32),
        ],
    )(s_j, ln_g.reshape(1, FEAT), ln_b.reshape(1, FEAT),
      Wq_e, Wq_o, Wkv_e, Wkv_o)


def _gather_piece(off, size, q, kv, nbrs_chunks):
    mesh = plsc.VectorSubcoreMesh(core_axis_name="c", subcore_axis_name="s")
    call = functools.partial(
        pl.kernel,
        mesh=mesh,
        compiler_params=pltpu.CompilerParams(needs_layout_passes=False),
        out_type=[
            jax.ShapeDtypeStruct((size, HF // 2), jnp.int32),
            jax.ShapeDtypeStruct((size, HF), jnp.int32),
        ],
        scratch_types=[
            pltpu.VMEM((2, _CH), jnp.int32),
            pltpu.VMEM((_CH, HF // 2), jnp.int32),
            pltpu.VMEM((_CH, HF), jnp.int32),
            pltpu.SemaphoreType.DMA,
        ],
    )(functools.partial(_sc_gather_body, off // _CH, size // _CH))
    return call(q, kv, nbrs_chunks)


def _edge_compute_piece(off, size, dist2, qi_p, kvj_p, Wdk_p, bdk_p, Wdv_p,
                        bdv_p, Wd_p, bd, prev_out):
    full = lambda shape: pl.BlockSpec(shape, lambda i: (0, 0))
    boff = off // _EDGE_BLK
    return pl.pallas_call(
        _edge_body,
        grid=(size // _EDGE_BLK,),
        in_specs=[
            pl.BlockSpec((_EDGE_BLK, 1), lambda i: (boff + i, 0)),
            pl.BlockSpec((_EDGE_BLK, HF // 2), lambda i: (i, 0)),
            pl.BlockSpec((_EDGE_BLK, HF), lambda i: (i, 0)),
            full((FEAT, HF)),
            full((1, HF)),
            full((FEAT, HF)),
            full((1, HF)),
            full((HF, 3 * FEAT)),
            full((1, 3 * FEAT)),
            pl.BlockSpec(memory_space=pl.ANY),
        ],
        out_specs=pl.BlockSpec((3, _EDGE_BLK, FEAT), lambda i: (0, boff + i, 0)),
        out_shape=jax.ShapeDtypeStruct((3, N_EDGES, FEAT), jnp.float32),
        input_output_aliases={} if prev_out is None else {9: 0},
    )(dist2, qi_p, kvj_p, Wdk_p, bdk_p.reshape(1, HF), Wdv_p,
      bdv_p.reshape(1, HF), Wd_p, bd.reshape(1, 3 * FEAT),
      jnp.zeros((8, 128), jnp.float32) if prev_out is None else prev_out)


def kernel(s_j, dist, nbrs, ln_g, ln_b, Wq, Wk, Wv, Wdk, bdk, Wdv, bdv, Wd, bd):
    Wkv = jnp.concatenate([Wk, Wv], axis=1)
    q, kv = _project_nodes(s_j, ln_g, ln_b, Wq[:, 0::2], Wq[:, 1::2],
                           Wkv[:, 0::2], Wkv[:, 1::2])
    nbrs_chunks = jnp.transpose(
        nbrs.astype(jnp.int32).reshape(_NCHUNK, _CH, 2), (0, 2, 1))
    perm = jnp.concatenate([jnp.arange(0, HF, 2), jnp.arange(1, HF, 2)])
    Wdk_p = jnp.zeros((FEAT, HF), jnp.float32).at[:N_RBF].set(Wdk)[:, perm]
    Wdv_p = jnp.zeros((FEAT, HF), jnp.float32).at[:N_RBF].set(Wdv)[:, perm]
    bdk_p = bdk[perm]
    bdv_p = bdv[perm]
    Wd_p = Wd[perm, :]
    dist2 = dist.reshape(N_EDGES, 1)

    offs = [sum(_PIECES[:i]) for i in range(len(_PIECES))]
    gathered = [_gather_piece(o, s, q, kv, nbrs_chunks)
                for o, s in zip(offs, _PIECES)]
    out = None
    for (o, s), (qi_p, kvj_p) in zip(zip(offs, _PIECES), gathered):
        out = _edge_compute_piece(o, s, dist2, qi_p, kvj_p, Wdk_p, bdk_p,
                                  Wdv_p, bdv_p, Wd_p, bd, out)
    return jnp.transpose(out, (1, 0, 2))


# speedup vs baseline: 2.5623x; 1.0437x over previous
import functools

import jax
import jax.numpy as jnp
from jax import lax
from jax.experimental import pallas as pl
from jax.experimental.pallas import tpu as pltpu
from jax.experimental.pallas import tpu_sc as plsc

N_NODES = 10000
FEAT = 128
NUM_HEADS = 2
HF = NUM_HEADS * FEAT
N_RBF = 20
CUTOFF = 5.0
N_EDGES = 160000

_NODE_BLK = 400
_EDGE_BLK = 1600
_CH = 128
_NW = 32
_NCHUNK = N_EDGES // _CH
_PIECES = (38400, 38400, 38400, 32000, 12800)
assert sum(_PIECES) == N_EDGES


def _silu(x):
    return x * jax.nn.sigmoid(x)


def _pack_rne(ev, od):
    be = lax.bitcast_convert_type(ev, jnp.uint32)
    bo = lax.bitcast_convert_type(od, jnp.uint32)
    re = (be + jnp.uint32(0x7FFF) + ((be >> 16) & jnp.uint32(1))) >> 16
    ro = (bo + jnp.uint32(0x7FFF) + ((bo >> 16) & jnp.uint32(1))) \
        & jnp.uint32(0xFFFF0000)
    return lax.bitcast_convert_type(re | ro, jnp.int32)


def _unpack(x):
    xb = lax.bitcast_convert_type(x, jnp.uint32)
    ev = lax.bitcast_convert_type(xb << 16, jnp.float32)
    od = lax.bitcast_convert_type(xb & jnp.uint32(0xFFFF0000), jnp.float32)
    return ev, od


def _node_body(s_ref, g_ref, b_ref, wqe_ref, wqo_ref, wkve_ref, wkvo_ref,
               q_ref, kv_ref):
    x = s_ref[...]
    mu = jnp.mean(x, axis=1, keepdims=True)
    xc = x - mu
    var = jnp.mean(xc * xc, axis=1, keepdims=True)
    xn = xc * lax.rsqrt(var + 1e-5) * g_ref[...] + b_ref[...]
    dot = lambda w: jnp.dot(xn, w, preferred_element_type=jnp.float32)
    q_ref[...] = _pack_rne(dot(wqe_ref[...]), dot(wqo_ref[...]))
    kv_ref[...] = _pack_rne(dot(wkve_ref[...]), dot(wkvo_ref[...]))


_NBIN = 128
_BIN_W = CUTOFF / (_NBIN - 1)


def _table_body(wdk_ref, bdk_ref, wdv_ref, bdv_ref, tk_ref, tv_ref):
    dcol = lax.broadcasted_iota(jnp.int32, (_NBIN, 1), 0).astype(jnp.float32) \
        * _BIN_W
    lane = lax.broadcasted_iota(jnp.int32, (1, FEAT), 1)
    width = CUTOFF / (N_RBF - 1)
    coeff = -0.5 / (width * width)
    diff = dcol - lane.astype(jnp.float32) * width
    rbf = jnp.where(lane < N_RBF, jnp.exp(coeff * diff * diff), 0.0)
    tk_ref[...] = _silu(
        jnp.dot(rbf, wdk_ref[...], preferred_element_type=jnp.float32)
        + bdk_ref[...])
    tv_ref[...] = _silu(
        jnp.dot(rbf, wdv_ref[...], preferred_element_type=jnp.float32)
        + bdv_ref[...])


def _edge_body(dist_ref, qi_ref, kvj_ref, tk_ref, tv_ref, wd_ref, bd_ref,
               prev_ref, out_ref):
    del prev_ref
    d = dist_ref[...]
    lane = lax.broadcasted_iota(jnp.int32, (1, _NBIN), 1)
    pos = d * (1.0 / _BIN_W) - lane.astype(jnp.float32)
    w_hat = jnp.maximum(0.0, 1.0 - jnp.abs(pos))
    dk = jnp.dot(w_hat, tk_ref[...], preferred_element_type=jnp.float32)
    dv = jnp.dot(w_hat, tv_ref[...], preferred_element_type=jnp.float32)
    qe, qo = _unpack(qi_ref[...])
    qi = jnp.concatenate([qe, qo], axis=1)
    kve, kvo = _unpack(kvj_ref[...])
    kj = jnp.concatenate([kve[:, :FEAT], kvo[:, :FEAT]], axis=1)
    vj = jnp.concatenate([kve[:, FEAT:], kvo[:, FEAT:]], axis=1)
    li = lax.broadcasted_iota(jnp.int32, (1, HF), 1)
    h0 = ((li // 64) % 2) == 0
    t = qi * kj * dk
    hl = lax.broadcasted_iota(jnp.int32, (HF, 2), 0)
    hc = lax.broadcasted_iota(jnp.int32, (HF, 2), 1)
    hsel = (((hl // 64) % 2) == hc).astype(jnp.float32)
    a = jnp.dot(t, hsel, preferred_element_type=jnp.float32)
    a0 = _silu(a[:, 0:1])
    a1 = _silu(a[:, 1:2])
    msg = vj * dv * jnp.where(h0, a0, a1)
    res = (jnp.dot(msg, wd_ref[...], preferred_element_type=jnp.float32)
           + bd_ref[...])
    out_ref[0, :, :] = res[:, :FEAT]
    out_ref[1, :, :] = res[:, FEAT:2 * FEAT]
    out_ref[2, :, :] = res[:, 2 * FEAT:]


def _sc_gather_body(chunk0, nchunks, q_hbm, kv_hbm, nbrs_hbm, qi_hbm, kvj_hbm,
                    nb_v, bq, bkv, sem):
    nc = 2
    wid = lax.axis_index("s") * nc + lax.axis_index("c")

    def body(t, carry):
        c = wid + t * _NW

        @pl.when(c < nchunks)
        def _():
            pltpu.sync_copy(nbrs_hbm.at[chunk0 + c], nb_v)
            c1 = pltpu.async_copy(q_hbm.at[nb_v.at[0]], bq, sem)
            c2 = pltpu.async_copy(kv_hbm.at[nb_v.at[1]], bkv, sem)
            c1.wait()
            c2.wait()
            lbase = c * _CH
            pltpu.sync_copy(bq, qi_hbm.at[pl.ds(lbase, _CH)])
            pltpu.sync_copy(bkv, kvj_hbm.at[pl.ds(lbase, _CH)])

        return carry

    iters = (nchunks + _NW - 1) // _NW
    lax.fori_loop(0, iters, body, None)


def _project_nodes(s_j, ln_g, ln_b, Wq_e, Wq_o, Wkv_e, Wkv_o):
    full = lambda shape: pl.BlockSpec(shape, lambda i: (0, 0))
    return pl.pallas_call(
        _node_body,
        grid=(N_NODES // _NODE_BLK,),
        in_specs=[
            pl.BlockSpec((_NODE_BLK, FEAT), lambda i: (i, 0)),
            full((1, FEAT)),
            full((1, FEAT)),
            full((FEAT, HF // 2)),
            full((FEAT, HF // 2)),
            full((FEAT, HF)),
            full((FEAT, HF)),
        ],
        out_specs=[
            pl.BlockSpec((_NODE_BLK, HF // 2), lambda i: (i, 0)),
            pl.BlockSpec((_NODE_BLK, HF), lambda i: (i, 0)),
        ],
        out_shape=[
            jax.ShapeDtypeStruct((N_NODES, HF // 2), jnp.int32),
            jax.ShapeDtypeStruct((N_NODES, HF), jnp.int32),
        ],
    )(s_j, ln_g.reshape(1, FEAT), ln_b.reshape(1, FEAT),
      Wq_e, Wq_o, Wkv_e, Wkv_o)


def _build_tables(Wdk_p, bdk_p, Wdv_p, bdv_p):
    full = lambda shape: pl.BlockSpec(shape, lambda: (0, 0))
    return pl.pallas_call(
        _table_body,
        in_specs=[
            full((FEAT, HF)),
            full((1, HF)),
            full((FEAT, HF)),
            full((1, HF)),
        ],
        out_specs=[full((_NBIN, HF)), full((_NBIN, HF))],
        out_shape=[jax.ShapeDtypeStruct((_NBIN, HF), jnp.float32)] * 2,
    )(Wdk_p, bdk_p.reshape(1, HF), Wdv_p, bdv_p.reshape(1, HF))


def _gather_piece(off, size, q, kv, nbrs_chunks):
    mesh = plsc.VectorSubcoreMesh(core_axis_name="c", subcore_axis_name="s")
    call = functools.partial(
        pl.kernel,
        mesh=mesh,
        compiler_params=pltpu.CompilerParams(needs_layout_passes=False),
        out_type=[
            jax.ShapeDtypeStruct((size, HF // 2), jnp.int32),
            jax.ShapeDtypeStruct((size, HF), jnp.int32),
        ],
        scratch_types=[
            pltpu.VMEM((2, _CH), jnp.int32),
            pltpu.VMEM((_CH, HF // 2), jnp.int32),
            pltpu.VMEM((_CH, HF), jnp.int32),
            pltpu.SemaphoreType.DMA,
        ],
    )(functools.partial(_sc_gather_body, off // _CH, size // _CH))
    return call(q, kv, nbrs_chunks)


def _edge_compute_piece(off, size, dist2, qi_p, kvj_p, tk, tv, Wd_p, bd,
                        prev_out):
    full = lambda shape: pl.BlockSpec(shape, lambda i: (0, 0))
    boff = off // _EDGE_BLK
    return pl.pallas_call(
        _edge_body,
        grid=(size // _EDGE_BLK,),
        in_specs=[
            pl.BlockSpec((_EDGE_BLK, 1), lambda i: (boff + i, 0)),
            pl.BlockSpec((_EDGE_BLK, HF // 2), lambda i: (i, 0)),
            pl.BlockSpec((_EDGE_BLK, HF), lambda i: (i, 0)),
            full((_NBIN, HF)),
            full((_NBIN, HF)),
            full((HF, 3 * FEAT)),
            full((1, 3 * FEAT)),
            pl.BlockSpec(memory_space=pl.ANY),
        ],
        out_specs=pl.BlockSpec((3, _EDGE_BLK, FEAT), lambda i: (0, boff + i, 0)),
        out_shape=jax.ShapeDtypeStruct((3, N_EDGES, FEAT), jnp.float32),
        input_output_aliases={} if prev_out is None else {7: 0},
    )(dist2, qi_p, kvj_p, tk, tv, Wd_p, bd.reshape(1, 3 * FEAT),
      jnp.zeros((8, 128), jnp.float32) if prev_out is None else prev_out)


def kernel(s_j, dist, nbrs, ln_g, ln_b, Wq, Wk, Wv, Wdk, bdk, Wdv, bdv, Wd, bd):
    Wkv = jnp.concatenate([Wk, Wv], axis=1)
    q, kv = _project_nodes(s_j, ln_g, ln_b, Wq[:, 0::2], Wq[:, 1::2],
                           Wkv[:, 0::2], Wkv[:, 1::2])
    nbrs_chunks = jnp.transpose(
        nbrs.astype(jnp.int32).reshape(_NCHUNK, _CH, 2), (0, 2, 1))
    perm = jnp.concatenate([jnp.arange(0, HF, 2), jnp.arange(1, HF, 2)])
    Wdk_p = jnp.zeros((FEAT, HF), jnp.float32).at[:N_RBF].set(Wdk)[:, perm]
    Wdv_p = jnp.zeros((FEAT, HF), jnp.float32).at[:N_RBF].set(Wdv)[:, perm]
    bdk_p = bdk[perm]
    bdv_p = bdv[perm]
    Wd_p = Wd[perm, :]
    dist2 = dist.reshape(N_EDGES, 1)

    tk, tv = _build_tables(Wdk_p, bdk_p, Wdv_p, bdv_p)
    offs = [sum(_PIECES[:i]) for i in range(len(_PIECES))]
    gathered = [_gather_piece(o, s, q, kv, nbrs_chunks)
                for o, s in zip(offs, _PIECES)]
    out = None
    for (o, s), (qi_p, kvj_p) in zip(zip(offs, _PIECES), gathered):
        out = _edge_compute_piece(o, s, dist2, qi_p, kvj_p, tk, tv,
                                  Wd_p, bd, out)
    return jnp.transpose(out, (1, 0, 2))
